# baseline (speedup 1.0000x reference)
import functools

import jax
import jax.numpy as jnp
from jax import lax
from jax.experimental import pallas as pl
from jax.experimental.pallas import tpu as pltpu
from jax.experimental.pallas import tpu_sc as plsc

_INFO = plsc.get_sparse_core_info()
_NC = _INFO.num_cores
_NS = _INFO.num_subcores
_SP_FRAC = 2
_SCH = 512
_TCH = 64


@functools.lru_cache(maxsize=None)
def _build(n, d):
    rows_c = n // _NC
    sp_rows = rows_c // _SP_FRAC
    st_rows = rows_c - sp_rows
    rows_t = st_rows // _NS
    assert rows_t % _TCH == 0 and sp_rows % _SCH == 0
    n_sch = sp_rows // _SCH
    n_tch = rows_t // _TCH
    mesh = plsc.VectorSubcoreMesh(core_axis_name="c", subcore_axis_name="s")

    @functools.partial(
        pl.kernel,
        mesh=mesh,
        out_type=jax.ShapeDtypeStruct((n, d), jnp.float32),
        scratch_types=[
            pltpu.VMEM_SHARED((sp_rows, d), jnp.float32),
            pltpu.VMEM((rows_t, d), jnp.float32),
            pltpu.VMEM((16,), jnp.int32),
            pltpu.SemaphoreType.DMA,
            pltpu.SemaphoreType.DMA,
            pltpu.SemaphoreType.DMA,
            pltpu.SemaphoreType.DMA,
            pltpu.SemaphoreType.DMA,
        ],
    )
    def k(table_hbm, clamp_hbm, out_hbm, sbuf, tbuf, clamp_ref,
          tisem, tosem, sisem, sosem, csem):
        cid = lax.axis_index("c")
        sid = lax.axis_index("s")
        cbase = cid * rows_c
        tbase = cbase + sp_rows + sid * rows_t

        tins = [
            pltpu.async_copy(
                table_hbm.at[pl.ds(tbase + j * _TCH, _TCH)],
                tbuf.at[pl.ds(j * _TCH, _TCH)],
                tisem,
            )
            for j in range(n_tch)
        ]

        @pl.when(sid == 0)
        def _spmem():
            cc = pltpu.async_copy(clamp_hbm, clamp_ref, csem)
            sins = [
                pltpu.async_copy(
                    table_hbm.at[pl.ds(cbase + j * _SCH, _SCH)],
                    sbuf.at[pl.ds(j * _SCH, _SCH)],
                    sisem,
                )
                for j in range(n_sch)
            ]
            souts = []
            for j in range(n_sch):
                sins[j].wait()
                souts.append(pltpu.async_copy(
                    sbuf.at[pl.ds(j * _SCH, _SCH)],
                    out_hbm.at[pl.ds(cbase + j * _SCH, _SCH)],
                    sosem,
                ))
            for c in souts:
                c.wait()
            cc.wait()

        touts = []
        for j in range(n_tch):
            tins[j].wait()
            touts.append(pltpu.async_copy(
                tbuf.at[pl.ds(j * _TCH, _TCH)],
                out_hbm.at[pl.ds(tbase + j * _TCH, _TCH)],
                tosem,
            ))
        for c in touts:
            c.wait()

        plsc.subcore_barrier()

        @pl.when(sid == 0)
        def _fix():
            clamp_s = clamp_ref[...][0]
            lo = jnp.maximum(clamp_s + 1, cbase)
            hi = cbase + rows_c

            def _body(r, carry):
                pltpu.sync_copy(
                    table_hbm.at[pl.ds(clamp_s, 1)],
                    out_hbm.at[pl.ds(r, 1)],
                )
                return carry

            lax.fori_loop(lo, hi, _body, 0)

    return k


def kernel(seq_len, table):
    n, d = table.shape
    clamp_val = jnp.maximum(jnp.asarray(seq_len, jnp.int32) - 1, 0)
    clamp = jnp.broadcast_to(clamp_val, (16,))
    return _build(n, d)(table, clamp)


# speedup vs baseline: 1.0591x; 1.0591x over previous
import functools

import jax
import jax.numpy as jnp
from jax import lax
from jax.experimental import pallas as pl
from jax.experimental.pallas import tpu as pltpu
from jax.experimental.pallas import tpu_sc as plsc

_NSC = 2
_CH = 256


@functools.lru_cache(maxsize=None)
def _build(n, d):
    rows_c = n // _NSC
    n_chunks = rows_c // _CH
    mesh = plsc.ScalarSubcoreMesh(axis_name="c", num_cores=_NSC)

    @functools.partial(
        pl.kernel,
        mesh=mesh,
        out_type=jax.ShapeDtypeStruct((n, d), jnp.float32),
        scratch_types=[
            pltpu.VMEM_SHARED((rows_c, d), jnp.float32),
            pltpu.SMEM((16,), jnp.int32),
            pltpu.SemaphoreType.DMA,
            pltpu.SemaphoreType.DMA,
            pltpu.SemaphoreType.DMA,
        ],
    )
    def k(table_hbm, clamp_hbm, out_hbm, buf, smem, isem, osem, csem):
        cid = lax.axis_index("c")
        base = cid * rows_c
        cc = pltpu.async_copy(clamp_hbm, smem, csem)
        ins = [
            pltpu.async_copy(
                table_hbm.at[pl.ds(base + j * _CH, _CH)],
                buf.at[pl.ds(j * _CH, _CH)],
                isem,
            )
            for j in range(n_chunks)
        ]
        outs = []
        for j in range(n_chunks):
            ins[j].wait()
            outs.append(pltpu.async_copy(
                buf.at[pl.ds(j * _CH, _CH)],
                out_hbm.at[pl.ds(base + j * _CH, _CH)],
                osem,
            ))
        for c in outs:
            c.wait()
        cc.wait()
        clamp_s = smem[0]
        lo = jnp.maximum(clamp_s + 1, base)
        hi = base + rows_c

        def _fix(r, carry):
            pltpu.sync_copy(
                table_hbm.at[pl.ds(clamp_s, 1)],
                out_hbm.at[pl.ds(r, 1)],
            )
            return carry

        lax.fori_loop(lo, hi, _fix, 0)

    return k


def kernel(seq_len, table):
    n, d = table.shape
    clamp_val = jnp.maximum(jnp.asarray(seq_len, jnp.int32) - 1, 0)
    clamp = jnp.broadcast_to(clamp_val, (16,))
    return _build(n, d)(table, clamp)


# speedup vs baseline: 1.0643x; 1.0050x over previous
import functools

import jax
import jax.numpy as jnp
from jax import lax
from jax.experimental import pallas as pl
from jax.experimental.pallas import tpu as pltpu
from jax.experimental.pallas import tpu_sc as plsc

_NSC = 2
_CH = 128


@functools.lru_cache(maxsize=None)
def _build(n, d):
    rows_c = n // _NSC
    n_chunks = rows_c // _CH
    mesh = plsc.ScalarSubcoreMesh(axis_name="c", num_cores=_NSC)

    @functools.partial(
        pl.kernel,
        mesh=mesh,
        out_type=jax.ShapeDtypeStruct((n, d), jnp.float32),
        scratch_types=[
            pltpu.VMEM_SHARED((rows_c, d), jnp.float32),
            pltpu.SMEM((16,), jnp.int32),
            pltpu.SemaphoreType.DMA,
            pltpu.SemaphoreType.DMA,
            pltpu.SemaphoreType.DMA,
        ],
    )
    def k(table_hbm, clamp_hbm, out_hbm, buf, smem, isem, osem, csem):
        cid = lax.axis_index("c")
        base = cid * rows_c
        cc = pltpu.async_copy(clamp_hbm, smem, csem)
        ins = [
            pltpu.async_copy(
                table_hbm.at[pl.ds(base + j * _CH, _CH)],
                buf.at[pl.ds(j * _CH, _CH)],
                isem,
            )
            for j in range(n_chunks)
        ]
        outs = []
        for j in range(n_chunks):
            ins[j].wait()
            outs.append(pltpu.async_copy(
                buf.at[pl.ds(j * _CH, _CH)],
                out_hbm.at[pl.ds(base + j * _CH, _CH)],
                osem,
            ))
        for c in outs:
            c.wait()
        cc.wait()
        clamp_s = smem[0]
        lo = jnp.maximum(clamp_s + 1, base)
        hi = base + rows_c

        def _fix(r, carry):
            pltpu.sync_copy(
                table_hbm.at[pl.ds(clamp_s, 1)],
                out_hbm.at[pl.ds(r, 1)],
            )
            return carry

        lax.fori_loop(lo, hi, _fix, 0)

    return k


def kernel(seq_len, table):
    n, d = table.shape
    clamp_val = jnp.maximum(jnp.asarray(seq_len, jnp.int32) - 1, 0)
    clamp = jnp.broadcast_to(clamp_val, (16,))
    return _build(n, d)(table, clamp)


# speedup vs baseline: 1.0688x; 1.0042x over previous
import functools

import jax
import jax.numpy as jnp
from jax import lax
from jax.experimental import pallas as pl
from jax.experimental.pallas import tpu as pltpu
from jax.experimental.pallas import tpu_sc as plsc

_NSC = 2
_CH = 64


@functools.lru_cache(maxsize=None)
def _build(n, d):
    rows_c = n // _NSC
    n_chunks = rows_c // _CH
    mesh = plsc.ScalarSubcoreMesh(axis_name="c", num_cores=_NSC)

    @functools.partial(
        pl.kernel,
        mesh=mesh,
        out_type=jax.ShapeDtypeStruct((n, d), jnp.float32),
        scratch_types=[
            pltpu.VMEM_SHARED((rows_c, d), jnp.float32),
            pltpu.SMEM((16,), jnp.int32),
            pltpu.SemaphoreType.DMA,
            pltpu.SemaphoreType.DMA,
            pltpu.SemaphoreType.DMA,
        ],
    )
    def k(table_hbm, clamp_hbm, out_hbm, buf, smem, isem, osem, csem):
        cid = lax.axis_index("c")
        base = cid * rows_c
        cc = pltpu.async_copy(clamp_hbm, smem, csem)
        ins = [
            pltpu.async_copy(
                table_hbm.at[pl.ds(base + j * _CH, _CH)],
                buf.at[pl.ds(j * _CH, _CH)],
                isem,
            )
            for j in range(n_chunks)
        ]
        outs = []
        for j in range(n_chunks):
            ins[j].wait()
            outs.append(pltpu.async_copy(
                buf.at[pl.ds(j * _CH, _CH)],
                out_hbm.at[pl.ds(base + j * _CH, _CH)],
                osem,
            ))
        for c in outs:
            c.wait()
        cc.wait()
        clamp_s = smem[0]
        lo = jnp.maximum(clamp_s + 1, base)
        hi = base + rows_c

        def _fix(r, carry):
            pltpu.sync_copy(
                table_hbm.at[pl.ds(clamp_s, 1)],
                out_hbm.at[pl.ds(r, 1)],
            )
            return carry

        lax.fori_loop(lo, hi, _fix, 0)

    return k


def kernel(seq_len, table):
    n, d = table.shape
    clamp_val = jnp.maximum(jnp.asarray(seq_len, jnp.int32) - 1, 0)
    clamp = jnp.broadcast_to(clamp_val, (16,))
    return _build(n, d)(table, clamp)
